# baseline (speedup 1.0000x reference)
import functools

import jax
import jax.numpy as jnp
from jax import lax
from jax.experimental import pallas as pl
from jax.experimental.pallas import tpu as pltpu
from jax.experimental.pallas import tpu_sc as plsc

_N = 8192
_R = 256
_G = _N // _R
_D0, _D1, _D2 = 256, 128, 64

_NC, _NS = 2, 16
_NW = _NC * _NS
_BW = _N // _NW
_CH = 128


def _gather_body(E0, E1, E2, i0, i1, i2, q0, q1, q2,
                 i0v, i1v, i2v, r0, r1, r2, s0, s1, s2):
    wid = lax.axis_index("s") * _NC + lax.axis_index("c")
    base = wid * _BW
    nrow = _BW // _CH
    pltpu.sync_copy(i0.at[pl.ds(wid * nrow, nrow)], i0v)
    pltpu.sync_copy(i1.at[pl.ds(wid * nrow, nrow)], i1v)
    pltpu.sync_copy(i2.at[pl.ds(wid * nrow, nrow)], i2v)
    c00 = pltpu.async_copy(E0.at[i0v.at[0]], r0, s0)
    c1 = [pltpu.async_copy(E1.at[i1v.at[k]],
                           r1.at[pl.ds(k * _CH, _CH)], s1)
          for k in range(nrow)]
    c2 = [pltpu.async_copy(E2.at[i2v.at[k]],
                           r2.at[pl.ds(k * _CH, _CH)], s2)
          for k in range(nrow)]
    c00.wait()
    pltpu.sync_copy(r0, q0.at[pl.ds(base, _CH)])
    c01 = pltpu.async_copy(E0.at[i0v.at[1]], r0, s0)
    for c in c1:
        c.wait()
    pltpu.sync_copy(r1, q1.at[pl.ds(base, _BW)])
    for c in c2:
        c.wait()
    pltpu.sync_copy(r2, q2.at[pl.ds(base, _BW)])
    c01.wait()
    pltpu.sync_copy(r0, q0.at[pl.ds(base + _CH, _CH)])


def _sc_gather_fn():
    return functools.partial(
        pl.kernel,
        mesh=plsc.VectorSubcoreMesh(core_axis_name="c", subcore_axis_name="s"),
        out_type=[
            jax.ShapeDtypeStruct((_N, _D0), jnp.float32),
            jax.ShapeDtypeStruct((_N, _D1), jnp.float32),
            jax.ShapeDtypeStruct((_N, 128), jnp.float32),
        ],
        scratch_types=[
            pltpu.VMEM((_BW // _CH, _CH), jnp.int32),
            pltpu.VMEM((_BW // _CH, _CH), jnp.int32),
            pltpu.VMEM((_BW // _CH, _CH), jnp.int32),
            pltpu.VMEM((_CH, _D0), jnp.float32),
            pltpu.VMEM((_BW, _D1), jnp.float32),
            pltpu.VMEM((_BW, 128), jnp.float32),
            pltpu.SemaphoreType.DMA,
            pltpu.SemaphoreType.DMA,
            pltpu.SemaphoreType.DMA,
        ],
    )(_gather_body)


def _dec_body(q0, q1, q2, anchor, Wd0, bd0, Wd1, bd1, Wd2, bd2, recon_ref):
    bf = lambda a: a.astype(jnp.bfloat16).astype(jnp.float32)
    comb = bf(jnp.concatenate([q0[...], q1[...], q2[...]], axis=1))
    d = jnp.maximum(jnp.dot(comb, Wd0[...]) + bd0[...], 0.0)
    d = bf(jnp.maximum(jnp.dot(d, Wd1[...]) + bd1[...], 0.0))
    recon_ref[...] = jnp.dot(d, Wd2[...]) + bd2[...]


def _full(shape):
    return pl.BlockSpec(shape, lambda i: (0,) * len(shape))


def _rows(shape):
    return pl.BlockSpec(shape, lambda i: (i,) + (0,) * (len(shape) - 1))


def _decode(q0, q1, q2, anchor, Wd0, bd0, Wd1, bd1, Wd2, bd2):
    in_specs = [_rows((_R, _D0)), _rows((_R, _D1)), _rows((_R, _D2)),
                _rows((_R, 768))]
    ws = [Wd0, bd0, Wd1, bd1, Wd2, bd2]
    in_specs += [_full(w.shape) for w in ws]
    return pl.pallas_call(
        _dec_body,
        grid=(_G,),
        in_specs=in_specs,
        out_specs=_rows((_R, 768)),
        out_shape=jax.ShapeDtypeStruct((_N, 768), jnp.float32),
    )(q0, q1, q2, anchor, *ws)


def kernel(x, We0, be0, We1, be1, Wh0, bh0, Wh1, bh1, Wh2, bh2,
           Wp0a, bp0a, Wp0b, bp0b, Wp1a, bp1a, Wp1b, bp1b,
           Wp2a, bp2a, Wp2b, bp2b, E0, E1, E2, Wd0, bd0, Wd1, bd1, Wd2, bd2):
    h = jax.nn.relu(x @ We0 + be0)
    h = jax.nn.relu(h @ We1 + be1)
    feats = [h @ Wh0 + bh0, h @ Wh1 + bh1, h @ Wh2 + bh2]
    projs = [(Wp0a, bp0a, Wp0b, bp0b), (Wp1a, bp1a, Wp1b, bp1b),
             (Wp2a, bp2a, Wp2b, bp2b)]
    Es = [E0, E1, E2]
    total_loss = 0.0
    quantized = []
    indices = []
    for f, (Wa, ba, Wb, bb), E in zip(feats, projs, Es):
        p = jax.nn.relu(f @ Wa + ba) @ Wb + bb
        d = (jnp.sum(p * p, axis=1, keepdims=True) + jnp.sum(E * E, axis=1)
             - 2.0 * (p @ E.T))
        idx = jnp.argmin(d, axis=1)
        q = jnp.take(E, idx, axis=0)
        e_loss = jnp.mean((jax.lax.stop_gradient(q) - p) ** 2)
        q_loss = jnp.mean((q - jax.lax.stop_gradient(p)) ** 2)
        total_loss = total_loss + q_loss + 0.25 * e_loss
        q_st = p + jax.lax.stop_gradient(q - p)
        quantized.append(q_st)
        indices.append(idx)
    comb = jnp.concatenate(quantized, axis=-1)
    dh = jax.nn.relu(comb @ Wd0 + bd0)
    dh = jax.nn.relu(dh @ Wd1 + bd1)
    anchor = dh @ Wd2 + bd2

    i0, i1, i2 = indices
    bE0, bE1, bE2, bi0, bi1, bi2, anchor = lax.optimization_barrier(
        (E0, E1, E2, i0, i1, i2, anchor))
    E2p = jnp.pad(bE2, ((0, 0), (0, 128 - _D2)))
    q0, q1, q2p = _sc_gather_fn()(
        bE0, bE1, E2p,
        bi0.reshape(_N // _CH, _CH), bi1.reshape(_N // _CH, _CH),
        bi2.reshape(_N // _CH, _CH))
    r2c = lambda b: b.reshape(1, -1)
    recon = _decode(q0, q1, q2p[:, :_D2], anchor,
                    Wd0, r2c(bd0), Wd1, r2c(bd1), Wd2, r2c(bd2))
    return recon, total_loss, i0, i1, i2


# speedup vs baseline: 1.1031x; 1.1031x over previous
import functools

import jax
import jax.numpy as jnp
from jax import lax
from jax.experimental import pallas as pl
from jax.experimental.pallas import tpu as pltpu
from jax.experimental.pallas import tpu_sc as plsc

_N = 8192
_R = 256
_G = _N // _R
_D0, _D1, _D2 = 256, 128, 64

_NC, _NS = 2, 16
_NW = _NC * _NS
_BW = _N // _NW
_CH = 128


def _gather_body(E0, E1, E2, i0, i1, i2, q0, q1, q2,
                 i0v, i1v, i2v, r0, r1, r2, s0, s1, s2):
    wid = lax.axis_index("s") * _NC + lax.axis_index("c")
    base = wid * _BW
    nrow = _BW // _CH
    pltpu.sync_copy(i0.at[pl.ds(wid * nrow, nrow)], i0v)
    pltpu.sync_copy(i1.at[pl.ds(wid * nrow, nrow)], i1v)
    pltpu.sync_copy(i2.at[pl.ds(wid * nrow, nrow)], i2v)
    c00 = pltpu.async_copy(E0.at[i0v.at[0]], r0, s0)
    c1 = [pltpu.async_copy(E1.at[i1v.at[k]],
                           r1.at[pl.ds(k * _CH, _CH)], s1)
          for k in range(nrow)]
    c2 = [pltpu.async_copy(E2.at[i2v.at[k]],
                           r2.at[pl.ds(k * _CH, _CH)], s2)
          for k in range(nrow)]
    c00.wait()
    pltpu.sync_copy(r0, q0.at[pl.ds(base, _CH)])
    c01 = pltpu.async_copy(E0.at[i0v.at[1]], r0, s0)
    for c in c1:
        c.wait()
    pltpu.sync_copy(r1, q1.at[pl.ds(base, _BW)])
    for c in c2:
        c.wait()
    pltpu.sync_copy(r2, q2.at[pl.ds(base, _BW)])
    c01.wait()
    pltpu.sync_copy(r0, q0.at[pl.ds(base + _CH, _CH)])


def _sc_gather_fn():
    return functools.partial(
        pl.kernel,
        mesh=plsc.VectorSubcoreMesh(core_axis_name="c", subcore_axis_name="s"),
        out_type=[
            jax.ShapeDtypeStruct((_N, _D0), jnp.float32),
            jax.ShapeDtypeStruct((_N, _D1), jnp.float32),
            jax.ShapeDtypeStruct((_N, 128), jnp.float32),
        ],
        scratch_types=[
            pltpu.VMEM((_BW // _CH, _CH), jnp.int32),
            pltpu.VMEM((_BW // _CH, _CH), jnp.int32),
            pltpu.VMEM((_BW // _CH, _CH), jnp.int32),
            pltpu.VMEM((_CH, _D0), jnp.float32),
            pltpu.VMEM((_BW, _D1), jnp.float32),
            pltpu.VMEM((_BW, 128), jnp.float32),
            pltpu.SemaphoreType.DMA,
            pltpu.SemaphoreType.DMA,
            pltpu.SemaphoreType.DMA,
        ],
    )(_gather_body)


def _dec_body(q0, q1, q2, anchor, Wd0, bd0, Wd1, bd1, Wd2, bd2, recon_ref):
    bf = lambda a: a.astype(jnp.bfloat16).astype(jnp.float32)
    comb = bf(jnp.concatenate([q0[...], q1[...], q2[...]], axis=1))
    d = jnp.maximum(jnp.dot(comb, Wd0[...]) + bd0[...], 0.0)
    d = bf(jnp.maximum(jnp.dot(d, Wd1[...]) + bd1[...], 0.0))
    recon_ref[...] = jnp.dot(d, Wd2[...]) + bd2[...]


def _full(shape):
    return pl.BlockSpec(shape, lambda i: (0,) * len(shape))


def _rows(shape):
    return pl.BlockSpec(shape, lambda i: (i,) + (0,) * (len(shape) - 1))


def _decode(q0, q1, q2, anchor, Wd0, bd0, Wd1, bd1, Wd2, bd2):
    in_specs = [_rows((_R, _D0)), _rows((_R, _D1)), _rows((_R, _D2)),
                _rows((_R, 448))]
    ws = [Wd0, bd0, Wd1, bd1, Wd2, bd2]
    in_specs += [_full(w.shape) for w in ws]
    return pl.pallas_call(
        _dec_body,
        grid=(_G,),
        in_specs=in_specs,
        out_specs=_rows((_R, 768)),
        out_shape=jax.ShapeDtypeStruct((_N, 768), jnp.float32),
    )(q0, q1, q2, anchor, *ws)


def kernel(x, We0, be0, We1, be1, Wh0, bh0, Wh1, bh1, Wh2, bh2,
           Wp0a, bp0a, Wp0b, bp0b, Wp1a, bp1a, Wp1b, bp1b,
           Wp2a, bp2a, Wp2b, bp2b, E0, E1, E2, Wd0, bd0, Wd1, bd1, Wd2, bd2):
    h = jax.nn.relu(x @ We0 + be0)
    h = jax.nn.relu(h @ We1 + be1)
    feats = [h @ Wh0 + bh0, h @ Wh1 + bh1, h @ Wh2 + bh2]
    projs = [(Wp0a, bp0a, Wp0b, bp0b), (Wp1a, bp1a, Wp1b, bp1b),
             (Wp2a, bp2a, Wp2b, bp2b)]
    Es = [E0, E1, E2]
    total_loss = 0.0
    quantized = []
    indices = []
    for f, (Wa, ba, Wb, bb), E in zip(feats, projs, Es):
        p = jax.nn.relu(f @ Wa + ba) @ Wb + bb
        d = (jnp.sum(p * p, axis=1, keepdims=True) + jnp.sum(E * E, axis=1)
             - 2.0 * (p @ E.T))
        idx = jnp.argmin(d, axis=1)
        q = jnp.take(E, idx, axis=0)
        e_loss = jnp.mean((jax.lax.stop_gradient(q) - p) ** 2)
        q_loss = jnp.mean((q - jax.lax.stop_gradient(p)) ** 2)
        total_loss = total_loss + q_loss + 0.25 * e_loss
        q_st = p + jax.lax.stop_gradient(q - p)
        quantized.append(q_st)
        indices.append(idx)
    anchor = jnp.concatenate(quantized, axis=-1)

    i0, i1, i2 = indices
    bE0, bE1, bE2, bi0, bi1, bi2, anchor = lax.optimization_barrier(
        (E0, E1, E2, i0, i1, i2, anchor))
    E2p = jnp.pad(bE2, ((0, 0), (0, 128 - _D2)))
    q0, q1, q2p = _sc_gather_fn()(
        bE0, bE1, E2p,
        bi0.reshape(_N // _CH, _CH), bi1.reshape(_N // _CH, _CH),
        bi2.reshape(_N // _CH, _CH))
    r2c = lambda b: b.reshape(1, -1)
    recon = _decode(q0, q1, q2p[:, :_D2], anchor,
                    Wd0, r2c(bd0), Wd1, r2c(bd1), Wd2, r2c(bd2))
    return recon, total_loss, i0, i1, i2
